# Initial kernel scaffold; baseline (speedup 1.0000x reference)
import jax
import jax.numpy as jnp
from jax.experimental import pallas as pl


def kernel(inputs, W0, W1, idx):
    raise NotImplementedError("write your pallas kernel here")


# speedup vs baseline: 1.5877x; 1.5877x over previous
import functools

import jax
import jax.numpy as jnp
from jax import lax
from jax.experimental import pallas as pl
from jax.experimental.pallas import tpu as pltpu
from jax.experimental.pallas import tpu_sc as plsc

EMBED = 256
NCODES = 16384
NVECS = 16384
COMMIT = 0.25

BM = 1024
BN = 2048
MT = NVECS // BM
NT = NCODES // BN
W1_TILE = 8192 // BN


def _argmin_body(idx_sref, xn_ref, cn_ref, ji_ref, x2_ref, c_ref, out_ref,
                 mmin, marg):
    j = pl.program_id(1)
    xb = x2_ref[...].astype(jnp.bfloat16)
    cb = c_ref[...].astype(jnp.bfloat16)
    dot2 = lax.dot_general(xb, cb, (((1,), (1,)), ((), ())),
                           preferred_element_type=jnp.float32)
    d = (xn_ref[...] + cn_ref[...]) - dot2
    m = jnp.min(d, axis=1, keepdims=True)
    argf = jnp.min(jnp.where(d == m, ji_ref[...], jnp.float32(3e7)),
                   axis=1, keepdims=True)
    arg = argf.astype(jnp.int32)

    @pl.when(j == 0)
    def _():
        mmin[...] = m
        marg[...] = arg

    skip = jnp.logical_and(idx_sref[0] == 0, j >= W1_TILE)

    @pl.when(jnp.logical_and(j > 0, jnp.logical_not(skip)))
    def _():
        better = m < mmin[...]
        marg[...] = jnp.where(better, arg, marg[...])
        mmin[...] = jnp.where(better, m, mmin[...])

    @pl.when(j == NT - 1)
    def _():
        out_ref[0] = marg[...]


def _argmin_call(idx_arr, xn, cn, ji, flat, codes):
    grid_spec = pltpu.PrefetchScalarGridSpec(
        num_scalar_prefetch=1,
        grid=(MT, NT),
        in_specs=[
            pl.BlockSpec((BM, 1), lambda i, j, *_: (i, 0)),
            pl.BlockSpec((1, BN), lambda i, j, *_: (0, j)),
            pl.BlockSpec((1, BN), lambda i, j, *_: (0, j)),
            pl.BlockSpec((BM, EMBED), lambda i, j, *_: (i, 0)),
            pl.BlockSpec((BN, EMBED), lambda i, j, *_: (j, 0)),
        ],
        out_specs=pl.BlockSpec((1, BM, 1), lambda i, j, *_: (i, 0, 0)),
        scratch_shapes=[
            pltpu.VMEM((BM, 1), jnp.float32),
            pltpu.VMEM((BM, 1), jnp.int32),
        ],
    )
    out = pl.pallas_call(
        _argmin_body,
        grid_spec=grid_spec,
        out_shape=jax.ShapeDtypeStruct((MT, BM, 1), jnp.int32),
        compiler_params=pltpu.CompilerParams(
            dimension_semantics=("arbitrary", "arbitrary")),
    )(idx_arr, xn, cn, ji, flat, codes)
    return out.reshape(NVECS)


_SC_NC = 2
_SC_NS = 16
_SC_NW = _SC_NC * _SC_NS
_ROWS_PER_W = NVECS // _SC_NW
_GCHUNK = 128


def _sc_gather_body(codes_hbm, idx_hbm, out_hbm, idx_v, rows_v, sem):
    c = lax.axis_index("c")
    s = lax.axis_index("s")
    wid = s * _SC_NC + c
    base = wid * _ROWS_PER_W
    pltpu.sync_copy(idx_hbm.at[pl.ds(base, _ROWS_PER_W)], idx_v)

    @pl.loop(0, _ROWS_PER_W // _GCHUNK)
    def _(k):
        off = pl.multiple_of(k * _GCHUNK, _GCHUNK)
        pltpu.async_copy(codes_hbm.at[idx_v.at[pl.ds(off, _GCHUNK)]],
                         rows_v, sem).wait()
        pltpu.sync_copy(rows_v, out_hbm.at[pl.ds(base + off, _GCHUNK)])


def _sc_gather(codes, enc):
    mesh = plsc.VectorSubcoreMesh(core_axis_name="c", subcore_axis_name="s")
    fn = pl.kernel(
        _sc_gather_body,
        mesh=mesh,
        out_type=jax.ShapeDtypeStruct((NVECS, EMBED), jnp.float32),
        scratch_types=[
            pltpu.VMEM((_ROWS_PER_W,), jnp.int32),
            pltpu.VMEM((_GCHUNK, EMBED), jnp.float32),
            pltpu.SemaphoreType.DMA,
        ],
    )
    return fn(codes, enc)


BM2 = 2048
MT2 = NVECS // BM2
_NELEM = float(NVECS * EMBED)


def _final_body(x_ref, q_ref, idxm_ref, qout_ref, loss_ref, perp_ref, acc):
    step = pl.program_id(0)
    x = x_ref[...]
    q = q_ref[...]
    qout_ref[...] = x + (q - x)
    part = jnp.sum((q - x) ** 2)

    @pl.when(step == 0)
    def _():
        acc[0, 0] = 0.0

    acc[0, 0] += part

    @pl.when(step == MT2 - 1)
    def _():
        idxv = idxm_ref[...]
        hi = lax.shift_right_logical(idxv, 7)
        lo = jnp.bitwise_and(idxv, 127)
        ih = lax.broadcasted_iota(jnp.int32, (NVECS, 128), 1)
        ehi = (hi == ih).astype(jnp.bfloat16)
        elo = (lo == ih).astype(jnp.bfloat16)
        counts = lax.dot_general(ehi, elo, (((0,), (0,)), ((), ())),
                                 preferred_element_type=jnp.float32)
        p = counts * (1.0 / NVECS)
        ent = -jnp.sum(p * jnp.log(p + 1e-10))
        perp_ref[...] = jnp.exp(ent).reshape(1, 1)
        mean_sq = acc[0, 0] * (1.0 / _NELEM)
        loss_ref[...] = (mean_sq + COMMIT * mean_sq).reshape(1, 1)


def _final_call(flat, qflat, idxm):
    out = pl.pallas_call(
        _final_body,
        grid=(MT2,),
        in_specs=[
            pl.BlockSpec((BM2, EMBED), lambda i: (i, 0)),
            pl.BlockSpec((BM2, EMBED), lambda i: (i, 0)),
            pl.BlockSpec((NVECS, 1), lambda i: (0, 0)),
        ],
        out_specs=[
            pl.BlockSpec((BM2, EMBED), lambda i: (i, 0)),
            pl.BlockSpec((1, 1), lambda i: (0, 0)),
            pl.BlockSpec((1, 1), lambda i: (0, 0)),
        ],
        out_shape=[
            jax.ShapeDtypeStruct((NVECS, EMBED), jnp.float32),
            jax.ShapeDtypeStruct((1, 1), jnp.float32),
            jax.ShapeDtypeStruct((1, 1), jnp.float32),
        ],
        scratch_shapes=[pltpu.SMEM((1, 1), jnp.float32)],
        compiler_params=pltpu.CompilerParams(
            dimension_semantics=("arbitrary",)),
    )(flat, qflat, idxm)
    return out


def kernel(inputs, W0, W1, idx):
    x = jnp.transpose(inputs, (0, 2, 3, 1))
    input_shape = x.shape
    flat = x.reshape(-1, EMBED)
    codes = jnp.concatenate([W0, W1], axis=0)
    xn = jnp.sum(flat ** 2, axis=1, keepdims=True)
    cn = jnp.sum(codes ** 2, axis=1).reshape(1, NCODES)
    idx_arr = jnp.asarray(idx, jnp.int32).reshape(1)
    ji = jnp.arange(NCODES, dtype=jnp.float32).reshape(1, NCODES)

    enc = _argmin_call(idx_arr, xn, cn, ji, flat + flat, codes)
    qflat = _sc_gather(codes, enc)
    qout_flat, loss11, perp11 = _final_call(flat, qflat, enc.reshape(NVECS, 1))

    quant = jnp.transpose(qout_flat.reshape(input_shape), (0, 3, 1, 2))
    return quant, loss11.reshape(()), perp11.reshape(())


# speedup vs baseline: 1.7032x; 1.0728x over previous
import functools

import jax
import jax.numpy as jnp
from jax import lax
from jax.experimental import pallas as pl
from jax.experimental.pallas import tpu as pltpu
from jax.experimental.pallas import tpu_sc as plsc

EMBED = 256
NCODES = 16384
NVECS = 16384
COMMIT = 0.25

BM = 1024
BN = 4096
MT = NVECS // BM
NT = NCODES // BN
W1_TILE = 8192 // BN


def _argmin_body(idx_sref, xn_ref, cn_ref, ji_ref, x2_ref, c_ref, out_ref,
                 mmin, marg):
    j = pl.program_id(0)
    i = pl.program_id(1)
    rows = pl.ds(i * BM, BM)
    xb = x2_ref[...].astype(jnp.bfloat16)
    cb = c_ref[...].astype(jnp.bfloat16)
    dot2 = lax.dot_general(xb, cb, (((1,), (1,)), ((), ())),
                           preferred_element_type=jnp.float32)
    d = (xn_ref[...] + cn_ref[...]) - dot2
    m = jnp.min(d, axis=1, keepdims=True)
    argf = jnp.min(jnp.where(d == m, ji_ref[...], jnp.float32(3e7)),
                   axis=1, keepdims=True)
    arg = argf.astype(jnp.int32)

    @pl.when(j == 0)
    def _():
        mmin[rows, :] = m
        marg[rows, :] = arg

    skip = jnp.logical_and(idx_sref[0] == 0, j >= W1_TILE)

    @pl.when(jnp.logical_and(j > 0, jnp.logical_not(skip)))
    def _():
        better = m < mmin[rows, :]
        marg[rows, :] = jnp.where(better, arg, marg[rows, :])
        mmin[rows, :] = jnp.where(better, m, mmin[rows, :])

    @pl.when(j == NT - 1)
    def _():
        out_ref[0] = marg[rows, :]


def _argmin_call(idx_arr, xn, cn, ji, flat, codes):
    grid_spec = pltpu.PrefetchScalarGridSpec(
        num_scalar_prefetch=1,
        grid=(NT, MT),
        in_specs=[
            pl.BlockSpec((BM, 1), lambda j, i, *_: (i, 0)),
            pl.BlockSpec((1, BN), lambda j, i, *_: (0, j)),
            pl.BlockSpec((1, BN), lambda j, i, *_: (0, j)),
            pl.BlockSpec((BM, EMBED), lambda j, i, *_: (i, 0)),
            pl.BlockSpec((BN, EMBED), lambda j, i, *_: (j, 0)),
        ],
        out_specs=pl.BlockSpec((1, BM, 1), lambda j, i, *_: (i, 0, 0)),
        scratch_shapes=[
            pltpu.VMEM((NVECS, 1), jnp.float32),
            pltpu.VMEM((NVECS, 1), jnp.int32),
        ],
    )
    out = pl.pallas_call(
        _argmin_body,
        grid_spec=grid_spec,
        out_shape=jax.ShapeDtypeStruct((MT, BM, 1), jnp.int32),
        compiler_params=pltpu.CompilerParams(
            dimension_semantics=("arbitrary", "arbitrary")),
    )(idx_arr, xn, cn, ji, flat, codes)
    return out.reshape(NVECS)


_SC_NC = 2
_SC_NS = 16
_SC_NW = _SC_NC * _SC_NS
_ROWS_PER_W = NVECS // _SC_NW
_GCHUNK = 128


def _sc_gather_body(codes_hbm, idx_hbm, out_hbm, idx_v, rows_v, sem):
    c = lax.axis_index("c")
    s = lax.axis_index("s")
    wid = s * _SC_NC + c
    base = wid * _ROWS_PER_W
    pltpu.sync_copy(idx_hbm.at[pl.ds(base, _ROWS_PER_W)], idx_v)

    @pl.loop(0, _ROWS_PER_W // _GCHUNK)
    def _(k):
        off = pl.multiple_of(k * _GCHUNK, _GCHUNK)
        pltpu.async_copy(codes_hbm.at[idx_v.at[pl.ds(off, _GCHUNK)]],
                         rows_v, sem).wait()
        pltpu.sync_copy(rows_v, out_hbm.at[pl.ds(base + off, _GCHUNK)])


def _sc_gather(codes, enc):
    mesh = plsc.VectorSubcoreMesh(core_axis_name="c", subcore_axis_name="s")
    fn = pl.kernel(
        _sc_gather_body,
        mesh=mesh,
        out_type=jax.ShapeDtypeStruct((NVECS, EMBED), jnp.float32),
        scratch_types=[
            pltpu.VMEM((_ROWS_PER_W,), jnp.int32),
            pltpu.VMEM((_GCHUNK, EMBED), jnp.float32),
            pltpu.SemaphoreType.DMA,
        ],
    )
    return fn(codes, enc)


BM2 = 2048
MT2 = NVECS // BM2
_NELEM = float(NVECS * EMBED)


def _final_body(x_ref, q_ref, idxm_ref, qout_ref, loss_ref, perp_ref, acc):
    step = pl.program_id(0)
    x = x_ref[...]
    q = q_ref[...]
    qout_ref[...] = x + (q - x)
    part = jnp.sum((q - x) ** 2)

    @pl.when(step == 0)
    def _():
        acc[0, 0] = 0.0

    acc[0, 0] += part

    @pl.when(step == MT2 - 1)
    def _():
        idxv = idxm_ref[...]
        hi = lax.shift_right_logical(idxv, 7)
        lo = jnp.bitwise_and(idxv, 127)
        ih = lax.broadcasted_iota(jnp.int32, (NVECS, 128), 1)
        ehi = (hi == ih).astype(jnp.bfloat16)
        elo = (lo == ih).astype(jnp.bfloat16)
        counts = lax.dot_general(ehi, elo, (((0,), (0,)), ((), ())),
                                 preferred_element_type=jnp.float32)
        p = counts * (1.0 / NVECS)
        ent = -jnp.sum(p * jnp.log(p + 1e-10))
        perp_ref[...] = jnp.exp(ent).reshape(1, 1)
        mean_sq = acc[0, 0] * (1.0 / _NELEM)
        loss_ref[...] = (mean_sq + COMMIT * mean_sq).reshape(1, 1)


def _final_call(flat, qflat, idxm):
    out = pl.pallas_call(
        _final_body,
        grid=(MT2,),
        in_specs=[
            pl.BlockSpec((BM2, EMBED), lambda i: (i, 0)),
            pl.BlockSpec((BM2, EMBED), lambda i: (i, 0)),
            pl.BlockSpec((NVECS, 1), lambda i: (0, 0)),
        ],
        out_specs=[
            pl.BlockSpec((BM2, EMBED), lambda i: (i, 0)),
            pl.BlockSpec((1, 1), lambda i: (0, 0)),
            pl.BlockSpec((1, 1), lambda i: (0, 0)),
        ],
        out_shape=[
            jax.ShapeDtypeStruct((NVECS, EMBED), jnp.float32),
            jax.ShapeDtypeStruct((1, 1), jnp.float32),
            jax.ShapeDtypeStruct((1, 1), jnp.float32),
        ],
        scratch_shapes=[pltpu.SMEM((1, 1), jnp.float32)],
        compiler_params=pltpu.CompilerParams(
            dimension_semantics=("arbitrary",)),
    )(flat, qflat, idxm)
    return out


def kernel(inputs, W0, W1, idx):
    x = jnp.transpose(inputs, (0, 2, 3, 1))
    input_shape = x.shape
    flat = x.reshape(-1, EMBED)
    codes = jnp.concatenate([W0, W1], axis=0)
    xn = jnp.sum(flat ** 2, axis=1, keepdims=True)
    cn = jnp.sum(codes ** 2, axis=1).reshape(1, NCODES)
    idx_arr = jnp.asarray(idx, jnp.int32).reshape(1)
    ji = jnp.arange(NCODES, dtype=jnp.float32).reshape(1, NCODES)

    enc = _argmin_call(idx_arr, xn, cn, ji, flat + flat, codes)
    qflat = _sc_gather(codes, enc)
    qout_flat, loss11, perp11 = _final_call(flat, qflat, enc.reshape(NVECS, 1))

    quant = jnp.transpose(qout_flat.reshape(input_shape), (0, 3, 1, 2))
    return quant, loss11.reshape(()), perp11.reshape(())


# speedup vs baseline: 1.7265x; 1.0137x over previous
import functools

import jax
import jax.numpy as jnp
from jax import lax
from jax.experimental import pallas as pl
from jax.experimental.pallas import tpu as pltpu
from jax.experimental.pallas import tpu_sc as plsc

EMBED = 256
NCODES = 16384
NVECS = 16384
COMMIT = 0.25

BM = 1024
BN = 4096
MT = NVECS // BM
NT = NCODES // BN
W1_TILE = 8192 // BN


def _argmin_body(idx_sref, xn_ref, cn_ref, ji_ref, x2_ref, c_ref, out_ref,
                 mmin, marg):
    j = pl.program_id(0)
    i = pl.program_id(1)
    rows = pl.ds(i * BM, BM)
    xb = x2_ref[...].astype(jnp.bfloat16)
    cb = c_ref[...].astype(jnp.bfloat16)
    dot2 = lax.dot_general(xb, cb, (((1,), (1,)), ((), ())),
                           preferred_element_type=jnp.float32)
    d = (xn_ref[...] + cn_ref[...]) - dot2
    m = jnp.min(d, axis=1, keepdims=True)
    argf = jnp.min(jnp.where(d == m, ji_ref[...], jnp.float32(3e7)),
                   axis=1, keepdims=True)
    arg = argf.astype(jnp.int32)

    @pl.when(j == 0)
    def _():
        mmin[rows, :] = m
        marg[rows, :] = arg

    skip = jnp.logical_and(idx_sref[0] == 0, j >= W1_TILE)

    @pl.when(jnp.logical_and(j > 0, jnp.logical_not(skip)))
    def _():
        better = m < mmin[rows, :]
        marg[rows, :] = jnp.where(better, arg, marg[rows, :])
        mmin[rows, :] = jnp.where(better, m, mmin[rows, :])

    @pl.when(j == NT - 1)
    def _():
        out_ref[0] = marg[rows, :]


def _argmin_call(idx_arr, xn, cn, ji, flat, codes):
    grid_spec = pltpu.PrefetchScalarGridSpec(
        num_scalar_prefetch=1,
        grid=(NT, MT),
        in_specs=[
            pl.BlockSpec((BM, 1), lambda j, i, *_: (i, 0)),
            pl.BlockSpec((1, BN), lambda j, i, *_: (0, j)),
            pl.BlockSpec((1, BN), lambda j, i, *_: (0, j)),
            pl.BlockSpec((BM, EMBED), lambda j, i, *_: (i, 0)),
            pl.BlockSpec((BN, EMBED), lambda j, i, *_: (j, 0)),
        ],
        out_specs=pl.BlockSpec((1, BM, 1), lambda j, i, *_: (i, 0, 0)),
        scratch_shapes=[
            pltpu.VMEM((NVECS, 1), jnp.float32),
            pltpu.VMEM((NVECS, 1), jnp.int32),
        ],
    )
    out = pl.pallas_call(
        _argmin_body,
        grid_spec=grid_spec,
        out_shape=jax.ShapeDtypeStruct((MT, BM, 1), jnp.int32),
        compiler_params=pltpu.CompilerParams(
            dimension_semantics=("arbitrary", "arbitrary")),
    )(idx_arr, xn, cn, ji, flat, codes)
    return out.reshape(NVECS)


_SC_NC = 2
_SC_NS = 16
_SC_NW = _SC_NC * _SC_NS
_ROWS_PER_W = NVECS // _SC_NW
_GCHUNK = 128


def _sc_gather_body(codes_hbm, idx_hbm, out_hbm, idx_v, rows_v, sem):
    c = lax.axis_index("c")
    s = lax.axis_index("s")
    wid = s * _SC_NC + c
    base = wid * _ROWS_PER_W
    pltpu.sync_copy(idx_hbm.at[pl.ds(base, _ROWS_PER_W)], idx_v)

    @pl.loop(0, _ROWS_PER_W // _GCHUNK)
    def _(k):
        off = pl.multiple_of(k * _GCHUNK, _GCHUNK)
        pltpu.async_copy(codes_hbm.at[idx_v.at[pl.ds(off, _GCHUNK)]],
                         rows_v, sem).wait()
        pltpu.sync_copy(rows_v, out_hbm.at[pl.ds(base + off, _GCHUNK)])


def _sc_gather(codes, enc):
    mesh = plsc.VectorSubcoreMesh(core_axis_name="c", subcore_axis_name="s")
    fn = pl.kernel(
        _sc_gather_body,
        mesh=mesh,
        out_type=jax.ShapeDtypeStruct((NVECS, EMBED), jnp.float32),
        scratch_types=[
            pltpu.VMEM((_ROWS_PER_W,), jnp.int32),
            pltpu.VMEM((_GCHUNK, EMBED), jnp.float32),
            pltpu.SemaphoreType.DMA,
        ],
    )
    return fn(codes, enc)


BM2 = 2048
MT2 = NVECS // BM2
_NELEM = float(NVECS * EMBED)


def _hist_body(idxm_ref, perp_ref):
    idxv = idxm_ref[...]
    hi = lax.shift_right_logical(idxv, 7)
    lo = jnp.bitwise_and(idxv, 127)
    ih = lax.broadcasted_iota(jnp.int32, (NVECS, 128), 1)
    ehi = (hi == ih).astype(jnp.bfloat16)
    elo = (lo == ih).astype(jnp.bfloat16)
    counts = lax.dot_general(ehi, elo, (((0,), (0,)), ((), ())),
                             preferred_element_type=jnp.float32)
    p = counts * (1.0 / NVECS)
    ent = -jnp.sum(p * jnp.log(p + 1e-10))
    perp_ref[...] = jnp.exp(ent).reshape(1, 1)


def _hist_call(idxm):
    return pl.pallas_call(
        _hist_body,
        out_shape=jax.ShapeDtypeStruct((1, 1), jnp.float32),
    )(idxm)


def _final_body(x_ref, q_ref, qout_ref, loss_ref, acc):
    step = pl.program_id(0)
    x = x_ref[...]
    q = q_ref[...]
    qout_ref[...] = x + (q - x)
    part = jnp.sum((q - x) ** 2)

    @pl.when(step == 0)
    def _():
        acc[0, 0] = 0.0

    acc[0, 0] += part

    @pl.when(step == MT2 - 1)
    def _():
        mean_sq = acc[0, 0] * (1.0 / _NELEM)
        loss_ref[...] = (mean_sq + COMMIT * mean_sq).reshape(1, 1)


def _final_call(flat, qflat):
    out = pl.pallas_call(
        _final_body,
        grid=(MT2,),
        in_specs=[
            pl.BlockSpec((BM2, EMBED), lambda i: (i, 0)),
            pl.BlockSpec((BM2, EMBED), lambda i: (i, 0)),
        ],
        out_specs=[
            pl.BlockSpec((BM2, EMBED), lambda i: (i, 0)),
            pl.BlockSpec((1, 1), lambda i: (0, 0)),
        ],
        out_shape=[
            jax.ShapeDtypeStruct((NVECS, EMBED), jnp.float32),
            jax.ShapeDtypeStruct((1, 1), jnp.float32),
        ],
        scratch_shapes=[pltpu.SMEM((1, 1), jnp.float32)],
        compiler_params=pltpu.CompilerParams(
            dimension_semantics=("arbitrary",)),
    )(flat, qflat)
    return out


def kernel(inputs, W0, W1, idx):
    x = jnp.transpose(inputs, (0, 2, 3, 1))
    input_shape = x.shape
    flat = x.reshape(-1, EMBED)
    codes = jnp.concatenate([W0, W1], axis=0)
    xn = jnp.sum(flat ** 2, axis=1, keepdims=True)
    cn = jnp.sum(codes ** 2, axis=1).reshape(1, NCODES)
    idx_arr = jnp.asarray(idx, jnp.int32).reshape(1)
    ji = jnp.arange(NCODES, dtype=jnp.float32).reshape(1, NCODES)

    enc = _argmin_call(idx_arr, xn, cn, ji, flat + flat, codes)
    qflat = _sc_gather(codes, enc)
    perp11 = _hist_call(enc.reshape(NVECS, 1))
    qout_flat, loss11 = _final_call(flat, qflat)

    quant = jnp.transpose(qout_flat.reshape(input_shape), (0, 3, 1, 2))
    return quant, loss11.reshape(()), perp11.reshape(())
